# Initial kernel scaffold; baseline (speedup 1.0000x reference)
import functools

import jax
import jax.numpy as jnp
import numpy as np
from jax.experimental import pallas as pl

ALPHA = 1.0
BETA = 1.0
N_TDA = 16
Q = 2
NN_SIZE = 5
RADIUS = 0.07
H = 0.03
EPS = 1e-12
BIG = 1e9

_T = 256

_IU = np.triu_indices(N_TDA, k=1)
_NPAIR = _IU[0].shape[0]
_PAD = 128


def _sel(idx):
    m = np.zeros((_PAD, N_TDA), dtype=np.float32)
    m[np.arange(_NPAIR), idx] = 1.0
    return m


_SEL_A = _sel(_IU[0])
_SEL_B = _sel(_IU[1])
_SEL_AT = np.ascontiguousarray(_SEL_A.T)
_SEL_BT = np.ascontiguousarray(_SEL_B.T)


def _sorted_pair_dists(p16, p16t, sela, selat, selb, selbt):
    f32 = jnp.float32
    pa = jnp.dot(sela, p16, preferred_element_type=f32)
    pb = jnp.dot(selb, p16, preferred_element_type=f32)
    d2c = jnp.sum((pa - pb) ** 2, axis=1, keepdims=True)
    vcol = jnp.sqrt(jnp.maximum(d2c, EPS))
    riota = jax.lax.broadcasted_iota(jnp.int32, (_PAD, 1), 0)
    vcol = jnp.where(riota < _NPAIR, vcol, BIG)
    par = jnp.dot(p16t, selat, preferred_element_type=f32)
    pbr = jnp.dot(p16t, selbt, preferred_element_type=f32)
    d2r = jnp.sum((par - pbr) ** 2, axis=0, keepdims=True)
    vrow = jnp.sqrt(jnp.maximum(d2r, EPS))
    ciota = jax.lax.broadcasted_iota(jnp.int32, (1, _PAD), 1)
    vrow = jnp.where(ciota < _NPAIR, vrow, BIG)
    ii = jax.lax.broadcasted_iota(jnp.int32, (_PAD, _PAD), 0)
    jj = jax.lax.broadcasted_iota(jnp.int32, (_PAD, _PAD), 1)
    less = jnp.where(vrow < vcol, 1.0, 0.0)
    tie = jnp.where((vrow == vcol) & (jj < ii), 1.0, 0.0)
    rank = jnp.sum(less + tie, axis=1, keepdims=True)
    kio = jax.lax.broadcasted_iota(jnp.float32, (1, _PAD), 1)
    onehot = jnp.where(rank == kio, 1.0, 0.0)
    return jnp.sum(onehot * vcol, axis=0, keepdims=True)


def _loss_kernel(pred_tile_ref, predt_ref, gtt_ref, gt16_ref, pcd_ref,
                 sela_ref, selat_ref, selb_ref, selbt_ref,
                 emd_ref, rep_ref, tda_ref):
    b = pl.program_id(0)
    i = pl.program_id(1)
    B = pl.num_programs(0)
    NT = pl.num_programs(1)
    N = NT * _T

    @pl.when((b == 0) & (i == 0))
    def _init():
        emd_ref[0, 0] = 0.0
        rep_ref[0, 0] = 0.0
        tda_ref[0, 0] = 0.0

    pt = pred_tile_ref[0]
    pf = predt_ref[0]
    gf = gtt_ref[0]

    def pair_d2(full):
        acc = None
        for c in range(3):
            pc = pt[:, c:c + 1]
            fc = full[c:c + 1, :]
            t = (pc - fc) ** 2
            acc = t if acc is None else acc + t
        return acc

    d_pg = pair_d2(gf)
    d_pp = pair_d2(pf)

    mn = jnp.min(d_pg, axis=1, keepdims=True)
    emd_part = jnp.sum(mn) * (100.0 / (3.0 * N * B)) / pcd_ref[0, 0]
    emd_ref[0, 0] += emd_part

    ciota = jax.lax.broadcasted_iota(jnp.int32, (_T, N), 1)
    riota = jax.lax.broadcasted_iota(jnp.int32, (_T, N), 0)
    d_pp = jnp.where(ciota == riota + i * _T, BIG, d_pp)
    rep_acc = 0.0
    for k in range(NN_SIZE - 1):
        m = jnp.min(d_pp, axis=1, keepdims=True)
        d2c = jnp.maximum(m, EPS)
        dist = jnp.sqrt(d2c)
        w = jnp.exp(-d2c / (H * H))
        rep_acc = rep_acc + jnp.sum((RADIUS - dist) * w)
        if k < NN_SIZE - 2:
            ismin = d_pp == m
            idxm = jnp.min(jnp.where(ismin, ciota, N), axis=1, keepdims=True)
            d_pp = jnp.where(ciota == idxm, BIG, d_pp)
    rep_ref[0, 0] += rep_acc / (B * N * (NN_SIZE - 1))

    @pl.when(i == 0)
    def _tda():
        p16 = pt[:N_TDA, :]
        p16t = pf[:, :N_TDA]
        g16 = gt16_ref[0]
        g16t = gf[:, :N_TDA]
        sp = _sorted_pair_dists(p16, p16t, sela_ref[...], selat_ref[...],
                                selb_ref[...], selbt_ref[...])
        sg = _sorted_pair_dists(g16, g16t, sela_ref[...], selat_ref[...],
                                selb_ref[...], selbt_ref[...])
        diff = sp - sg
        sumsq = jnp.sum(diff * diff)
        tda_ref[0, 0] += jnp.power(sumsq + EPS, 1.0 / Q) / B


@functools.partial(jax.jit, static_argnames=("interpret",))
def _run(pred, gt, pcd_radius, interpret=False):
    B, N, _ = pred.shape
    predt = jnp.swapaxes(pred, 1, 2)
    gtt = jnp.swapaxes(gt, 1, 2)
    grid = (B, N // _T)
    out_shape = [jax.ShapeDtypeStruct((1, 1), jnp.float32)] * 3
    scalar_spec = pl.BlockSpec((1, 1), lambda b, i: (0, 0))
    emd, rep, tda = pl.pallas_call(
        _loss_kernel,
        grid=grid,
        in_specs=[
            pl.BlockSpec((1, _T, 3), lambda b, i: (b, i, 0)),
            pl.BlockSpec((1, 3, N), lambda b, i: (b, 0, 0)),
            pl.BlockSpec((1, 3, N), lambda b, i: (b, 0, 0)),
            pl.BlockSpec((1, N_TDA, 3), lambda b, i: (b, 0, 0)),
            pl.BlockSpec((1, 1), lambda b, i: (b, 0)),
            pl.BlockSpec((_PAD, N_TDA), lambda b, i: (0, 0)),
            pl.BlockSpec((N_TDA, _PAD), lambda b, i: (0, 0)),
            pl.BlockSpec((_PAD, N_TDA), lambda b, i: (0, 0)),
            pl.BlockSpec((N_TDA, _PAD), lambda b, i: (0, 0)),
        ],
        out_specs=[scalar_spec, scalar_spec, scalar_spec],
        out_shape=out_shape,
        interpret=interpret,
    )(pred, predt, gtt, gt, pcd_radius,
      jnp.asarray(_SEL_A), jnp.asarray(_SEL_AT),
      jnp.asarray(_SEL_B), jnp.asarray(_SEL_BT))
    return (emd[0, 0], rep[0, 0] * ALPHA, tda[0, 0] * BETA)


def kernel(pred, gt, pcd_radius):
    return _run(pred, gt, pcd_radius)


# speedup vs baseline: 16.9759x; 16.9759x over previous
import functools

import jax
import jax.numpy as jnp
import numpy as np
from jax.experimental import pallas as pl

ALPHA = 1.0
BETA = 1.0
N_TDA = 16
Q = 2
NN_SIZE = 5
RADIUS = 0.07
H = 0.03
EPS = 1e-12
BIG = 1e9

_T = 256

_IU = np.triu_indices(N_TDA, k=1)
_NPAIR = _IU[0].shape[0]
_PAD = 128


def _sel(idx):
    m = np.zeros((_PAD, N_TDA), dtype=np.float32)
    m[np.arange(_NPAIR), idx] = 1.0
    return m


_SEL_A = _sel(_IU[0])
_SEL_B = _sel(_IU[1])


def _sorted_pair_dists(p16r, sela, selb):
    d2c = None
    for c in range(3):
        pa = jnp.sum(sela * p16r[c], axis=1, keepdims=True)
        pb = jnp.sum(selb * p16r[c], axis=1, keepdims=True)
        t = (pa - pb) ** 2
        d2c = t if d2c is None else d2c + t
    vcol = jnp.sqrt(jnp.maximum(d2c, EPS))
    riota = jax.lax.broadcasted_iota(jnp.int32, (_PAD, 1), 0)
    vcol = jnp.where(riota < _NPAIR, vcol, BIG)
    ii = jax.lax.broadcasted_iota(jnp.int32, (_PAD, _PAD), 0)
    jj = jax.lax.broadcasted_iota(jnp.int32, (_PAD, _PAD), 1)
    vcolb = jnp.broadcast_to(vcol, (_PAD, _PAD))
    vrow = jnp.sum(jnp.where(ii == jj, vcolb, 0.0), axis=0, keepdims=True)
    one = jnp.ones((_PAD, _PAD), jnp.int32)
    zero = jnp.zeros((_PAD, _PAD), jnp.int32)
    less = jnp.where(vrow < vcol, one, zero)
    tie = jnp.where((vrow == vcol) & (jj < ii), one, zero)
    rank = jnp.sum(less + tie, axis=1, keepdims=True)
    kio = jax.lax.broadcasted_iota(jnp.int32, (1, _PAD), 1)
    onehot = jnp.where(rank == kio, 1.0, 0.0)
    return jnp.sum(onehot * vcolb, axis=0, keepdims=True)


def _loss_kernel(ptile_ref, pfull_ref, gfull_ref,
                 pcx_ref, pcy_ref, pcz_ref,
                 prx_ref, pry_ref, prz_ref,
                 grx_ref, gry_ref, grz_ref,
                 pbb_ref, gbb_ref,
                 p16x_ref, p16y_ref, p16z_ref,
                 g16x_ref, g16y_ref, g16z_ref,
                 pcd_ref, sela_ref, selb_ref,
                 emd_ref, rep_ref, tda_ref):
    b = pl.program_id(0)
    i = pl.program_id(1)
    B = pl.num_programs(0)
    NT = pl.num_programs(1)
    N = NT * _T

    @pl.when((b == 0) & (i == 0))
    def _init():
        zero = jnp.zeros((1, 1), jnp.float32)
        emd_ref[...] = zero
        rep_ref[...] = zero
        tda_ref[...] = zero

    pt = ptile_ref[0]
    pc = [pcx_ref[0], pcy_ref[0], pcz_ref[0]]
    pr = [prx_ref[0], pry_ref[0], prz_ref[0]]
    gr = [grx_ref[0], gry_ref[0], grz_ref[0]]

    aa = jnp.sum(pt * pt, axis=1, keepdims=True)
    ciota = jax.lax.broadcasted_iota(jnp.int32, (_T, N), 1)

    def sel_matrix(full_ref, bb_ref):
        ab = jax.lax.dot_general(pt, full_ref[0], (((1,), (1,)), ((), ())),
                                 preferred_element_type=jnp.float32)
        return (aa - 2.0 * ab) + bb_ref[0]

    def exact_d2(rows):
        acc = None
        for c in range(3):
            t = (pc[c] - rows[c]) ** 2
            acc = t if acc is None else acc + t
        return acc

    def first_min_idx(d, m):
        return jnp.min(jnp.where(d == m, ciota, N), axis=1, keepdims=True)

    def value_at(d_exact, idx):
        return jnp.sum(jnp.where(ciota == idx, d_exact, 0.0),
                       axis=1, keepdims=True)

    d_pg_sel = sel_matrix(gfull_ref, gbb_ref)
    d_pg_exact = exact_d2(gr)
    m = jnp.min(d_pg_sel, axis=1, keepdims=True)
    idx = first_min_idx(d_pg_sel, m)
    val = value_at(d_pg_exact, idx)
    mnsum = jnp.sum(val, axis=0, keepdims=True)
    emd_ref[...] += mnsum * (100.0 / (3.0 * N * B)) / pcd_ref[0]

    d_pp_sel = sel_matrix(pfull_ref, pbb_ref)
    d_pp_exact = exact_d2(pr)
    rep_acc = jnp.zeros((1, 1), jnp.float32)
    for k in range(NN_SIZE):
        m = jnp.min(d_pp_sel, axis=1, keepdims=True)
        idxm = first_min_idx(d_pp_sel, m)
        if k > 0:
            val = value_at(d_pp_exact, idxm)
            d2c = jnp.maximum(val, EPS)
            dist = jnp.sqrt(d2c)
            w = jnp.exp(-d2c / (H * H))
            rep_acc = rep_acc + jnp.sum((RADIUS - dist) * w, axis=0,
                                        keepdims=True)
        if k < NN_SIZE - 1:
            d_pp_sel = jnp.where(ciota == idxm, BIG, d_pp_sel)
    rep_ref[...] += rep_acc / (B * N * (NN_SIZE - 1))

    @pl.when(i == 0)
    def _tda():
        p16r = [p16x_ref[0], p16y_ref[0], p16z_ref[0]]
        g16r = [g16x_ref[0], g16y_ref[0], g16z_ref[0]]
        sp = _sorted_pair_dists(p16r, sela_ref[...], selb_ref[...])
        sg = _sorted_pair_dists(g16r, sela_ref[...], selb_ref[...])
        diff = sp - sg
        sumsq = jnp.sum(diff * diff, axis=1, keepdims=True)
        tda_ref[...] += jnp.sqrt(sumsq + EPS) / B


@functools.partial(jax.jit, static_argnames=("interpret",))
def _run(pred, gt, pcd_radius, interpret=False):
    B, N, _ = pred.shape
    pcols = [pred[:, :, c:c + 1] for c in range(3)]
    prows = [jnp.swapaxes(p, 1, 2) for p in pcols]
    grows = [jnp.swapaxes(gt[:, :, c:c + 1], 1, 2) for c in range(3)]
    p16r = [r[:, :, :N_TDA] for r in prows]
    g16r = [r[:, :, :N_TDA] for r in grows]
    pbb = jnp.swapaxes(jnp.sum(pred * pred, axis=-1, keepdims=True), 1, 2)
    gbb = jnp.swapaxes(jnp.sum(gt * gt, axis=-1, keepdims=True), 1, 2)
    grid = (B, N // _T)
    tile_spec = pl.BlockSpec((1, _T, 3), lambda b, i: (b, i, 0))
    full_spec = pl.BlockSpec((1, N, 3), lambda b, i: (b, 0, 0))
    col_spec = pl.BlockSpec((1, _T, 1), lambda b, i: (b, i, 0))
    row_spec = pl.BlockSpec((1, 1, N), lambda b, i: (b, 0, 0))
    r16_spec = pl.BlockSpec((1, 1, N_TDA), lambda b, i: (b, 0, 0))
    sel_spec = pl.BlockSpec((_PAD, N_TDA), lambda b, i: (0, 0))
    scalar_spec = pl.BlockSpec((1, 1), lambda b, i: (0, 0))
    emd, rep, tda = pl.pallas_call(
        _loss_kernel,
        grid=grid,
        in_specs=[tile_spec, full_spec, full_spec] +
                 [col_spec] * 3 + [row_spec] * 6 + [row_spec] * 2 +
                 [r16_spec] * 6 +
                 [pl.BlockSpec((1, 1, 1), lambda b, i: (b, 0, 0))] +
                 [sel_spec] * 2,
        out_specs=[scalar_spec] * 3,
        out_shape=[jax.ShapeDtypeStruct((1, 1), jnp.float32)] * 3,
        interpret=interpret,
    )(pred, pred, gt, *pcols, *prows, *grows, pbb, gbb, *p16r, *g16r,
      pcd_radius.reshape(B, 1, 1),
      jnp.asarray(_SEL_A), jnp.asarray(_SEL_B))
    return (emd[0, 0], rep[0, 0] * ALPHA, tda[0, 0] * BETA)


def kernel(pred, gt, pcd_radius):
    return _run(pred, gt, pcd_radius)
